# baseline (speedup 1.0000x reference)
import functools

import jax
import jax.numpy as jnp
import numpy as np
from jax.experimental import pallas as pl
from jax.experimental.pallas import tpu as pltpu

GRID_X = 1
GRID_Y = 1
L_XY = [0.5, 1.0]
N_NEIGH = 100
K = (2 * GRID_X + 1) * (2 * GRID_Y + 1)
C_IN = 64
C_CAT = C_IN + 2
OUT_CH = 128
MLP_OUT = 512
N_PTS = 4096

_OFFS = []
for _l in L_XY:
    _dx = np.arange(-GRID_X, GRID_X + 1, dtype=np.float32) * _l
    _dy = np.arange(-GRID_Y, GRID_Y + 1, dtype=np.float32) * _l
    _OFFS.append(np.stack(np.meshgrid(_dx, _dy, indexing="ij"), axis=-1)
                 .reshape(-1, 2))

_N_ITERS = 24


def _body(pts_ref, feat_ref, ctr_ref, w2_ref, cb_ref, mlpw_ref, mlpb_ref,
          out_ref, *, mb):
    px = pts_ref[0, 0, :]
    py = pts_ref[0, 1, :]
    cx = ctr_ref[0, 0, 0, :]
    cy = ctr_ref[0, 0, 1, :]
    feat = feat_ref[0]

    nn = jnp.int32(N_NEIGH)
    ctr_pad = jnp.concatenate(
        [cx[:, None], cy[:, None], jnp.zeros((mb, C_IN), jnp.float32)],
        axis=1)
    scale_outs = []
    comb_center = None
    for s in range(len(L_XY)):
        off = _OFFS[s]
        ks = list(range(K)) if s == 0 else [k for k in range(K) if k != K // 2]
        d2_rows = []
        for k in ks:
            dx = (cx + float(off[k, 0]))[:, None] - px[None, :]
            dy = (cy + float(off[k, 1]))[:, None] - py[None, :]
            d2_rows.append(dx * dx + dy * dy)
        d2 = jnp.concatenate(d2_rows, axis=0)
        q = mb * len(ks)

        lo = jnp.zeros((q, 1), jnp.float32)
        hi = jnp.full((q, 1), 8.0, jnp.float32)
        thresh = jnp.full((q, 1), -1.0, jnp.float32)
        for _ in range(_N_ITERS):
            mid = (lo + hi) * 0.5
            cnt = jnp.sum((d2 <= mid).astype(jnp.int32), axis=1,
                          keepdims=True)
            hit = jnp.logical_and(cnt == nn, thresh < 0.0)
            thresh = jnp.where(hit, mid, thresh)
            ge = cnt >= nn
            lo = jnp.where(ge, lo, mid)
            hi = jnp.where(ge, mid, hi)
        thresh = jnp.where(thresh < 0.0, hi, thresh)

        recip = jax.lax.rsqrt(jnp.maximum(d2, 1e-16))
        w = jnp.where(d2 <= thresh, recip, 0.0)
        comb_e = jax.lax.dot(w, feat, precision=jax.lax.Precision.HIGHEST)
        comb = comb_e[:, :C_CAT] / comb_e[:, C_CAT:C_CAT + 1]
        blocks = {k: comb[i * mb:(i + 1) * mb, :] for i, k in enumerate(ks)}
        if s == 0:
            comb_center = blocks[K // 2]
        else:
            blocks[K // 2] = comb_center

        nf = jnp.concatenate(
            [blocks[k] - ctr_pad for k in range(K)],
            axis=1)
        o = jax.lax.dot(nf, w2_ref[...], precision=jax.lax.Precision.HIGHEST)
        o = jnp.maximum(o + cb_ref[0][None, :], 0.0)
        scale_outs.append(o)

    cat = jnp.concatenate(scale_outs, axis=1)
    rf = jax.lax.dot(cat, mlpw_ref[...], precision=jax.lax.Precision.HIGHEST)
    rf = jnp.maximum(rf + mlpb_ref[0][None, :], 0.0)
    out_ref[0] = rf


@functools.partial(jax.jit, static_argnames=("mb", "interpret"))
def _run(pts_t, featcat, ctr_t, w2, cb, mlpw, mlpb, *, mb, interpret=False):
    b, nblk, _, _ = ctr_t.shape
    grid = (b, nblk)
    return pl.pallas_call(
        functools.partial(_body, mb=mb),
        grid=grid,
        in_specs=[
            pl.BlockSpec((1, 2, N_PTS), lambda i, j: (i, 0, 0)),
            pl.BlockSpec((1, N_PTS, C_CAT + 1), lambda i, j: (i, 0, 0)),
            pl.BlockSpec((1, 1, 2, mb), lambda i, j: (i, j, 0, 0)),
            pl.BlockSpec((K * C_CAT, OUT_CH), lambda i, j: (0, 0)),
            pl.BlockSpec((1, OUT_CH), lambda i, j: (0, 0)),
            pl.BlockSpec((OUT_CH * len(L_XY), MLP_OUT), lambda i, j: (0, 0)),
            pl.BlockSpec((1, MLP_OUT), lambda i, j: (0, 0)),
        ],
        out_specs=pl.BlockSpec((1, mb, MLP_OUT), lambda i, j: (i, j, 0)),
        out_shape=jax.ShapeDtypeStruct((b, nblk * mb, MLP_OUT), jnp.float32),
        interpret=interpret,
    )(pts_t, featcat, ctr_t, w2, cb, mlpw, mlpb)


def kernel(xyz, feature, new_xyz, conv_w, conv_b, mlp_w, mlp_b,
           mb=128, interpret=False):
    pts_t = jnp.transpose(xyz[:, :, :2], (0, 2, 1))
    featcat = jnp.concatenate(
        [xyz[:, :, :2], feature,
         jnp.ones((xyz.shape[0], xyz.shape[1], 1), jnp.float32)],
        axis=-1)
    b, m = new_xyz.shape[0], new_xyz.shape[1]
    ctr_t = jnp.transpose(new_xyz[:, :, :2].reshape(b, m // mb, mb, 2),
                          (0, 1, 3, 2))
    w2 = jnp.transpose(conv_w, (2, 1, 0)).reshape(K * C_CAT, OUT_CH)
    mlpw = jnp.transpose(mlp_w)
    rf = _run(pts_t, featcat, ctr_t, w2, conv_b[None, :], mlpw,
              mlp_b[None, :], mb=mb, interpret=interpret)
    return (new_xyz, rf)


# speedup vs baseline: 1.0250x; 1.0250x over previous
import functools

import jax
import jax.numpy as jnp
import numpy as np
from jax.experimental import pallas as pl
from jax.experimental.pallas import tpu as pltpu

GRID_X = 1
GRID_Y = 1
L_XY = [0.5, 1.0]
N_NEIGH = 100
K = (2 * GRID_X + 1) * (2 * GRID_Y + 1)
C_IN = 64
C_CAT = C_IN + 2
OUT_CH = 128
MLP_OUT = 512
N_PTS = 4096

_OFFS = []
for _l in L_XY:
    _dx = np.arange(-GRID_X, GRID_X + 1, dtype=np.float32) * _l
    _dy = np.arange(-GRID_Y, GRID_Y + 1, dtype=np.float32) * _l
    _OFFS.append(np.stack(np.meshgrid(_dx, _dy, indexing="ij"), axis=-1)
                 .reshape(-1, 2))

_N_ITERS = 23


def _body(pts_ref, feat_ref, ctr_ref, w2_ref, cb_ref, mlpw_ref, mlpb_ref,
          out_ref, *, mb):
    px = pts_ref[0, 0, :]
    py = pts_ref[0, 1, :]
    cx = ctr_ref[0, 0, 0, :]
    cy = ctr_ref[0, 0, 1, :]
    feat = feat_ref[0]

    nn = jnp.int32(N_NEIGH)
    ctr_pad = jnp.concatenate(
        [cx[:, None], cy[:, None], jnp.zeros((mb, C_IN), jnp.float32)],
        axis=1)
    scale_outs = []
    comb_center = None
    for s in range(len(L_XY)):
        off = _OFFS[s]
        ks = list(range(K)) if s == 0 else [k for k in range(K) if k != K // 2]
        d2_rows = []
        for k in ks:
            dx = (cx + float(off[k, 0]))[:, None] - px[None, :]
            dy = (cy + float(off[k, 1]))[:, None] - py[None, :]
            d2_rows.append(dx * dx + dy * dy)
        d2 = jnp.concatenate(d2_rows, axis=0)
        q = mb * len(ks)

        lo = jnp.zeros((q, 1), jnp.float32)
        hi = jnp.full((q, 1), 8.0, jnp.float32)
        thresh = jnp.full((q, 1), -1.0, jnp.float32)
        for _ in range(_N_ITERS):
            mid = (lo + hi) * 0.5
            cnt = jnp.sum((d2 <= mid).astype(jnp.int32), axis=1,
                          keepdims=True)
            hit = jnp.logical_and(cnt == nn, thresh < 0.0)
            thresh = jnp.where(hit, mid, thresh)
            ge = cnt >= nn
            lo = jnp.where(ge, lo, mid)
            hi = jnp.where(ge, mid, hi)
        thresh = jnp.where(thresh < 0.0, hi, thresh)

        recip = jax.lax.rsqrt(jnp.maximum(d2, 1e-16))
        w = jnp.where(d2 <= thresh, recip, 0.0)
        comb_e = jax.lax.dot(w, feat, precision=jax.lax.Precision.HIGHEST)
        comb = comb_e[:, :C_CAT] / comb_e[:, C_CAT:C_CAT + 1]
        blocks = {k: comb[i * mb:(i + 1) * mb, :] for i, k in enumerate(ks)}
        if s == 0:
            comb_center = blocks[K // 2]
        else:
            blocks[K // 2] = comb_center

        nf = jnp.concatenate(
            [blocks[k] - ctr_pad for k in range(K)],
            axis=1)
        o = jax.lax.dot(nf, w2_ref[...], precision=jax.lax.Precision.HIGHEST)
        o = jnp.maximum(o + cb_ref[0][None, :], 0.0)
        scale_outs.append(o)

    cat = jnp.concatenate(scale_outs, axis=1)
    rf = jax.lax.dot(cat, mlpw_ref[...], precision=jax.lax.Precision.HIGHEST)
    rf = jnp.maximum(rf + mlpb_ref[0][None, :], 0.0)
    out_ref[0] = rf


@functools.partial(jax.jit, static_argnames=("mb", "interpret"))
def _run(pts_t, featcat, ctr_t, w2, cb, mlpw, mlpb, *, mb, interpret=False):
    b, nblk, _, _ = ctr_t.shape
    grid = (b, nblk)
    return pl.pallas_call(
        functools.partial(_body, mb=mb),
        grid=grid,
        in_specs=[
            pl.BlockSpec((1, 2, N_PTS), lambda i, j: (i, 0, 0)),
            pl.BlockSpec((1, N_PTS, C_CAT + 1), lambda i, j: (i, 0, 0)),
            pl.BlockSpec((1, 1, 2, mb), lambda i, j: (i, j, 0, 0)),
            pl.BlockSpec((K * C_CAT, OUT_CH), lambda i, j: (0, 0)),
            pl.BlockSpec((1, OUT_CH), lambda i, j: (0, 0)),
            pl.BlockSpec((OUT_CH * len(L_XY), MLP_OUT), lambda i, j: (0, 0)),
            pl.BlockSpec((1, MLP_OUT), lambda i, j: (0, 0)),
        ],
        out_specs=pl.BlockSpec((1, mb, MLP_OUT), lambda i, j: (i, j, 0)),
        out_shape=jax.ShapeDtypeStruct((b, nblk * mb, MLP_OUT), jnp.float32),
        interpret=interpret,
    )(pts_t, featcat, ctr_t, w2, cb, mlpw, mlpb)


def kernel(xyz, feature, new_xyz, conv_w, conv_b, mlp_w, mlp_b,
           mb=128, interpret=False):
    pts_t = jnp.transpose(xyz[:, :, :2], (0, 2, 1))
    featcat = jnp.concatenate(
        [xyz[:, :, :2], feature,
         jnp.ones((xyz.shape[0], xyz.shape[1], 1), jnp.float32)],
        axis=-1)
    b, m = new_xyz.shape[0], new_xyz.shape[1]
    ctr_t = jnp.transpose(new_xyz[:, :, :2].reshape(b, m // mb, mb, 2),
                          (0, 1, 3, 2))
    w2 = jnp.transpose(conv_w, (2, 1, 0)).reshape(K * C_CAT, OUT_CH)
    mlpw = jnp.transpose(mlp_w)
    rf = _run(pts_t, featcat, ctr_t, w2, conv_b[None, :], mlpw,
              mlp_b[None, :], mb=mb, interpret=interpret)
    return (new_xyz, rf)
